# baseline (speedup 1.0000x reference)
import functools

import jax
import jax.numpy as jnp
from jax import lax
from jax.experimental import pallas as pl
from jax.experimental.pallas import tpu as pltpu
from jax.experimental.pallas import tpu_sc as plsc

N = 10000
E = 320000
D = 128
EPS = 1e-5

NC = 2
NS = 16
K = 80
E_PER_CORE = E // NC
E_PER_SUB = E_PER_CORE // NS
N_CHUNKS = E_PER_SUB // K
NPAD = 10240
ROWS_PER_SUB = NPAD // NS

@functools.cache
def _build_deg_kernel():
    mesh = plsc.VectorSubcoreMesh(core_axis_name="c", subcore_axis_name="s")

    @functools.partial(
        pl.kernel,
        out_type=jax.ShapeDtypeStruct((NC, NPAD, D), jnp.float32),
        mesh=mesh,
        scratch_types=[
            pltpu.VMEM((N_CHUNKS, K), jnp.int32),
            pltpu.VMEM((K, D), jnp.float32),
            pltpu.SemaphoreType.DMA,
            pltpu.VMEM_SHARED((NPAD, D), jnp.float32),
        ],
    )
    def deg_kernel(dst_hbm, ones_hbm, zeros_hbm, out_hbm,
                   didx, ones_v, sem, acc):
        c = lax.axis_index("c")
        s = lax.axis_index("s")
        wid = c * NS + s
        pltpu.sync_copy(ones_hbm, ones_v)
        pltpu.sync_copy(zeros_hbm,
                        acc.at[pl.ds(s * ROWS_PER_SUB, ROWS_PER_SUB)])
        pltpu.sync_copy(dst_hbm.at[wid], didx)
        plsc.subcore_barrier()

        @pl.loop(0, N_CHUNKS)
        def _(i):
            pltpu.async_copy(ones_v, acc.at[didx.at[i]], sem, add=True)

        @pl.loop(0, N_CHUNKS)
        def _(i):
            pltpu.make_async_copy(ones_v, acc.at[didx.at[i]], sem).wait()

        plsc.subcore_barrier()
        pltpu.sync_copy(
            acc.at[pl.ds(s * ROWS_PER_SUB, ROWS_PER_SUB)],
            out_hbm.at[c, pl.ds(s * ROWS_PER_SUB, ROWS_PER_SUB)],
        )

    return deg_kernel


@functools.cache
def _build_agg_kernel():
    mesh = plsc.VectorSubcoreMesh(core_axis_name="c", subcore_axis_name="s")

    @functools.partial(
        pl.kernel,
        out_type=jax.ShapeDtypeStruct((NC, NPAD, D), jnp.float32),
        mesh=mesh,
        scratch_types=[
            pltpu.VMEM((N_CHUNKS * K,), jnp.int32),
            pltpu.VMEM((N_CHUNKS, K), jnp.int32),
            pltpu.VMEM((K, D), jnp.float32),
            pltpu.VMEM((K, D), jnp.float32),
            pltpu.SemaphoreType.DMA,
            pltpu.SemaphoreType.DMA,
            pltpu.SemaphoreType.DMA,
            pltpu.SemaphoreType.DMA,
            pltpu.VMEM_SHARED((NPAD, D), jnp.float32),
        ],
    )
    def agg_kernel(h2_hbm, src_hbm, dst_hbm, zeros_hbm, out_hbm,
                   sidx, didx, rows0, rows1, sem0, sem1, sems0, sems1, acc):
        c = lax.axis_index("c")
        s = lax.axis_index("s")
        wid = c * NS + s
        pltpu.sync_copy(zeros_hbm,
                        acc.at[pl.ds(s * ROWS_PER_SUB, ROWS_PER_SUB)])
        pltpu.sync_copy(src_hbm.at[pl.ds(wid * N_CHUNKS * K, N_CHUNKS * K)],
                        sidx)
        pltpu.sync_copy(dst_hbm.at[wid], didx)
        plsc.subcore_barrier()

        def gather_start(i, rows, sem):
            pltpu.async_copy(h2_hbm.at[sidx.at[pl.ds(i * K, K)]], rows, sem)

        def gather_wait(i, rows, sem):
            pltpu.make_async_copy(h2_hbm.at[sidx.at[pl.ds(i * K, K)]],
                                  rows, sem).wait()

        def scatter(i, rows):
            pltpu.sync_copy(rows, acc.at[didx.at[i]], add=True)

        def scatter_start(i, rows, sem):
            pltpu.async_copy(rows, acc.at[didx.at[i]], sem, add=True)

        def scatter_wait(i, rows, sem):
            pltpu.make_async_copy(rows, acc.at[didx.at[i]], sem).wait()

        gather_start(0, rows0, sem0)
        gather_wait(0, rows0, sem0)
        scatter_start(0, rows0, sems0)
        gather_start(1, rows1, sem1)

        @pl.loop(1, N_CHUNKS - 1, step=2)
        def _(j):
            gather_wait(j, rows1, sem1)
            scatter_start(j, rows1, sems1)
            scatter_wait(j - 1, rows0, sems0)
            gather_start(j + 1, rows0, sem0)
            gather_wait(j + 1, rows0, sem0)
            scatter_start(j + 1, rows0, sems0)
            scatter_wait(j, rows1, sems1)

            @pl.when(j < N_CHUNKS - 2)
            def _():
                gather_start(j + 2, rows1, sem1)

        scatter_wait(N_CHUNKS - 1, rows0, sems0)

        plsc.subcore_barrier()
        pltpu.sync_copy(
            acc.at[pl.ds(s * ROWS_PER_SUB, ROWS_PER_SUB)],
            out_hbm.at[c, pl.ds(s * ROWS_PER_SUB, ROWS_PER_SUB)],
        )

    return agg_kernel


def _deg_kernel(dst3, ones_k, zerosD):
    return _build_deg_kernel()(dst3, ones_k, zerosD)[:, :N]


def _agg_kernel(h2, src, dst3, zerosD):
    return _build_agg_kernel()(h2, src, dst3, zerosD)[:, :N]


BN = 1000


def _mm_prep_body(p0_ref, p1_ref, x_ref, w_ref, oh_ref, od_ref):
    deg = p0_ref[:, :1] + p1_ref[:, :1] + 1.0
    dv = jnp.broadcast_to(lax.rsqrt(deg), (BN, D))
    od_ref[...] = dv
    oh_ref[...] = jnp.dot(x_ref[...], w_ref[...],
                          preferred_element_type=jnp.float32) * dv


def _mm_prep(degp, x, w):
    blk = pl.BlockSpec((BN, D), lambda i: (i, 0))
    return pl.pallas_call(
        _mm_prep_body,
        grid=(N // BN,),
        in_specs=[blk, blk, blk, pl.BlockSpec((D, D), lambda i: (0, 0))],
        out_specs=[blk, blk],
        out_shape=[jax.ShapeDtypeStruct((N, D), jnp.float32),
                   jax.ShapeDtypeStruct((N, D), jnp.float32)],
    )(degp[0], degp[1], x, w)


def _epi_core(a0, a1, h2, dv, b, xraw):
    pre = dv * (a0 + a1 + h2) + b
    r = jnp.maximum(pre, 0.0)
    mu = jnp.mean(r, axis=-1, keepdims=True)
    var = jnp.mean((r - mu) ** 2, axis=-1, keepdims=True)
    ln = (r - mu) * lax.rsqrt(var + EPS)
    return ln + xraw


def _epi_mm_body(a0_ref, a1_ref, h2_ref, dv_ref, b_ref, xr_ref, w_ref,
                 ox_ref, oh_ref):
    xn = _epi_core(a0_ref[...], a1_ref[...], h2_ref[...], dv_ref[...],
                   b_ref[...], xr_ref[...])
    ox_ref[...] = xn
    oh_ref[...] = jnp.dot(xn, w_ref[...],
                          preferred_element_type=jnp.float32) * dv_ref[...]


def _epi_mm(acc, h2, dinv2d, b, xraw, w_next):
    blk = pl.BlockSpec((BN, D), lambda i: (i, 0))
    return pl.pallas_call(
        _epi_mm_body,
        grid=(N // BN,),
        in_specs=[blk, blk, blk, blk,
                  pl.BlockSpec((1, D), lambda i: (0, 0)), blk,
                  pl.BlockSpec((D, D), lambda i: (0, 0))],
        out_specs=[blk, blk],
        out_shape=[jax.ShapeDtypeStruct((N, D), jnp.float32),
                   jax.ShapeDtypeStruct((N, D), jnp.float32)],
    )(acc[0], acc[1], h2, dinv2d, b.reshape(1, D), xraw, w_next)


def _epi_body(a0_ref, a1_ref, h2_ref, dv_ref, b_ref, xr_ref, ox_ref):
    ox_ref[...] = _epi_core(a0_ref[...], a1_ref[...], h2_ref[...],
                            dv_ref[...], b_ref[...], xr_ref[...])


def _epi(acc, h2, dinv2d, b, xraw):
    blk = pl.BlockSpec((BN, D), lambda i: (i, 0))
    return pl.pallas_call(
        _epi_body,
        grid=(N // BN,),
        in_specs=[blk, blk, blk, blk,
                  pl.BlockSpec((1, D), lambda i: (0, 0)), blk],
        out_specs=blk,
        out_shape=jax.ShapeDtypeStruct((N, D), jnp.float32),
    )(acc[0], acc[1], h2, dinv2d, b.reshape(1, D), xraw)


def kernel(x, edge, W0, b0, W1, b1, W2, b2):
    edge = edge.astype(jnp.int32)
    src = edge[0]
    dst = edge[1]
    dst3 = dst.reshape(NC * NS, N_CHUNKS, K)
    ones_k = jnp.ones((K, D), jnp.float32)
    zerosD = jnp.zeros((ROWS_PER_SUB, D), jnp.float32)

    degp = _deg_kernel(dst3, ones_k, zerosD)
    h2, dinv2d = _mm_prep(degp, x, W0)
    acc = _agg_kernel(h2, src, dst3, zerosD)
    x1, h2 = _epi_mm(acc, h2, dinv2d, b0, x, W1)

    acc = _agg_kernel(h2, src, dst3, zerosD)
    x2, h2 = _epi_mm(acc, h2, dinv2d, b1, x1, W2)

    acc = _agg_kernel(h2, src, dst3, zerosD)
    return _epi(acc, h2, dinv2d, b2, x2)


# speedup vs baseline: 1.2040x; 1.2040x over previous
import functools

import jax
import jax.numpy as jnp
from jax import lax
from jax.experimental import pallas as pl
from jax.experimental.pallas import tpu as pltpu
from jax.experimental.pallas import tpu_sc as plsc

N = 10000
E = 320000
D = 128
EPS = 1e-5

NC = 2
NS = 16
K = 80
E_PER_CORE = E // NC
E_PER_SUB = E_PER_CORE // NS
N_CHUNKS = E_PER_SUB // K
NPAD = 10240
ROWS_PER_SUB = NPAD // NS

@functools.cache
def _build_deg_kernel():
    mesh = plsc.VectorSubcoreMesh(core_axis_name="c", subcore_axis_name="s")

    @functools.partial(
        pl.kernel,
        out_type=jax.ShapeDtypeStruct((NC, NPAD, D), jnp.float32),
        mesh=mesh,
        scratch_types=[
            pltpu.VMEM((N_CHUNKS, K), jnp.int32),
            pltpu.VMEM((K, D), jnp.float32),
            pltpu.SemaphoreType.DMA,
            pltpu.VMEM_SHARED((NPAD, D), jnp.float32),
        ],
    )
    def deg_kernel(dst_hbm, ones_hbm, zeros_hbm, out_hbm,
                   didx, ones_v, sem, acc):
        c = lax.axis_index("c")
        s = lax.axis_index("s")
        wid = c * NS + s
        pltpu.sync_copy(ones_hbm, ones_v)
        pltpu.sync_copy(zeros_hbm,
                        acc.at[pl.ds(s * ROWS_PER_SUB, ROWS_PER_SUB)])
        pltpu.sync_copy(dst_hbm.at[wid], didx)
        plsc.subcore_barrier()

        @pl.loop(0, N_CHUNKS)
        def _(i):
            pltpu.async_copy(ones_v, acc.at[didx.at[i]], sem, add=True)

        @pl.loop(0, N_CHUNKS)
        def _(i):
            pltpu.make_async_copy(ones_v, acc.at[didx.at[i]], sem).wait()

        plsc.subcore_barrier()
        pltpu.sync_copy(
            acc.at[pl.ds(s * ROWS_PER_SUB, ROWS_PER_SUB)],
            out_hbm.at[c, pl.ds(s * ROWS_PER_SUB, ROWS_PER_SUB)],
        )

    return deg_kernel


@functools.cache
def _build_agg_kernel():
    mesh = plsc.VectorSubcoreMesh(core_axis_name="c", subcore_axis_name="s")

    @functools.partial(
        pl.kernel,
        out_type=jax.ShapeDtypeStruct((NC, NPAD, D), jnp.float32),
        mesh=mesh,
        scratch_types=[
            pltpu.VMEM((N_CHUNKS * K,), jnp.int32),
            pltpu.VMEM((N_CHUNKS, K), jnp.int32),
            pltpu.VMEM((K, D), jnp.float32),
            pltpu.VMEM((K, D), jnp.float32),
            pltpu.SemaphoreType.DMA,
            pltpu.SemaphoreType.DMA,
            pltpu.VMEM_SHARED((NPAD, D), jnp.float32),
        ],
    )
    def agg_kernel(h2_hbm, src_hbm, dst_hbm, zeros_hbm, out_hbm,
                   sidx, didx, rows0, rows1, sem0, sem1, acc):
        c = lax.axis_index("c")
        s = lax.axis_index("s")
        wid = c * NS + s
        pltpu.sync_copy(zeros_hbm,
                        acc.at[pl.ds(s * ROWS_PER_SUB, ROWS_PER_SUB)])
        pltpu.sync_copy(src_hbm.at[pl.ds(wid * N_CHUNKS * K, N_CHUNKS * K)],
                        sidx)
        pltpu.sync_copy(dst_hbm.at[wid], didx)
        plsc.subcore_barrier()

        def gather_start(i, rows, sem):
            pltpu.async_copy(h2_hbm.at[sidx.at[pl.ds(i * K, K)]], rows, sem)

        def gather_wait(i, rows, sem):
            pltpu.make_async_copy(h2_hbm.at[sidx.at[pl.ds(i * K, K)]],
                                  rows, sem).wait()

        def scatter(i, rows):
            pltpu.sync_copy(rows, acc.at[didx.at[i]], add=True)

        gather_start(0, rows0, sem0)

        @pl.loop(0, N_CHUNKS - 1, step=2)
        def _(i):
            gather_start(i + 1, rows1, sem1)
            gather_wait(i, rows0, sem0)
            scatter(i, rows0)
            gather_start(i + 2, rows0, sem0)
            gather_wait(i + 1, rows1, sem1)
            scatter(i + 1, rows1)

        gather_wait(N_CHUNKS - 1, rows0, sem0)
        scatter(N_CHUNKS - 1, rows0)

        plsc.subcore_barrier()
        pltpu.sync_copy(
            acc.at[pl.ds(s * ROWS_PER_SUB, ROWS_PER_SUB)],
            out_hbm.at[c, pl.ds(s * ROWS_PER_SUB, ROWS_PER_SUB)],
        )

    return agg_kernel


def _deg_kernel(dst3, ones_k, zerosD):
    return _build_deg_kernel()(dst3, ones_k, zerosD)[:, :N]


def _agg_kernel(h2, src, dst3, zerosD):
    return _build_agg_kernel()(h2, src, dst3, zerosD)[:, :N]


BN = 1000


def _mm_prep_body(p0_ref, p1_ref, x_ref, w_ref, oh_ref, od_ref):
    deg = p0_ref[:, :1] + p1_ref[:, :1] + 1.0
    dv = jnp.broadcast_to(lax.rsqrt(deg), (BN, D))
    od_ref[...] = dv
    oh_ref[...] = jnp.dot(x_ref[...], w_ref[...],
                          preferred_element_type=jnp.float32) * dv


def _mm_prep(degp, x, w):
    blk = pl.BlockSpec((BN, D), lambda i: (i, 0))
    return pl.pallas_call(
        _mm_prep_body,
        grid=(N // BN,),
        in_specs=[blk, blk, blk, pl.BlockSpec((D, D), lambda i: (0, 0))],
        out_specs=[blk, blk],
        out_shape=[jax.ShapeDtypeStruct((N, D), jnp.float32),
                   jax.ShapeDtypeStruct((N, D), jnp.float32)],
    )(degp[0], degp[1], x, w)


def _epi_core(a0, a1, h2, dv, b, xraw):
    pre = dv * (a0 + a1 + h2) + b
    r = jnp.maximum(pre, 0.0)
    mu = jnp.mean(r, axis=-1, keepdims=True)
    var = jnp.mean((r - mu) ** 2, axis=-1, keepdims=True)
    ln = (r - mu) * lax.rsqrt(var + EPS)
    return ln + xraw


def _epi_mm_body(a0_ref, a1_ref, h2_ref, dv_ref, b_ref, xr_ref, w_ref,
                 ox_ref, oh_ref):
    xn = _epi_core(a0_ref[...], a1_ref[...], h2_ref[...], dv_ref[...],
                   b_ref[...], xr_ref[...])
    ox_ref[...] = xn
    oh_ref[...] = jnp.dot(xn, w_ref[...],
                          preferred_element_type=jnp.float32) * dv_ref[...]


def _epi_mm(acc, h2, dinv2d, b, xraw, w_next):
    blk = pl.BlockSpec((BN, D), lambda i: (i, 0))
    return pl.pallas_call(
        _epi_mm_body,
        grid=(N // BN,),
        in_specs=[blk, blk, blk, blk,
                  pl.BlockSpec((1, D), lambda i: (0, 0)), blk,
                  pl.BlockSpec((D, D), lambda i: (0, 0))],
        out_specs=[blk, blk],
        out_shape=[jax.ShapeDtypeStruct((N, D), jnp.float32),
                   jax.ShapeDtypeStruct((N, D), jnp.float32)],
    )(acc[0], acc[1], h2, dinv2d, b.reshape(1, D), xraw, w_next)


def _epi_body(a0_ref, a1_ref, h2_ref, dv_ref, b_ref, xr_ref, ox_ref):
    ox_ref[...] = _epi_core(a0_ref[...], a1_ref[...], h2_ref[...],
                            dv_ref[...], b_ref[...], xr_ref[...])


def _epi(acc, h2, dinv2d, b, xraw):
    blk = pl.BlockSpec((BN, D), lambda i: (i, 0))
    return pl.pallas_call(
        _epi_body,
        grid=(N // BN,),
        in_specs=[blk, blk, blk, blk,
                  pl.BlockSpec((1, D), lambda i: (0, 0)), blk],
        out_specs=blk,
        out_shape=jax.ShapeDtypeStruct((N, D), jnp.float32),
    )(acc[0], acc[1], h2, dinv2d, b.reshape(1, D), xraw)


def kernel(x, edge, W0, b0, W1, b1, W2, b2):
    edge = edge.astype(jnp.int32)
    src = edge[0]
    dst = edge[1]
    dst3 = dst.reshape(NC * NS, N_CHUNKS, K)
    ones_k = jnp.ones((K, D), jnp.float32)
    zerosD = jnp.zeros((ROWS_PER_SUB, D), jnp.float32)

    degp = _deg_kernel(dst3, ones_k, zerosD)
    h2, dinv2d = _mm_prep(degp, x, W0)
    acc = _agg_kernel(h2, src, dst3, zerosD)
    x1, h2 = _epi_mm(acc, h2, dinv2d, b0, x, W1)

    acc = _agg_kernel(h2, src, dst3, zerosD)
    x2, h2 = _epi_mm(acc, h2, dinv2d, b1, x1, W2)

    acc = _agg_kernel(h2, src, dst3, zerosD)
    return _epi(acc, h2, dinv2d, b2, x2)


# speedup vs baseline: 1.2042x; 1.0002x over previous
import functools

import jax
import jax.numpy as jnp
from jax import lax
from jax.experimental import pallas as pl
from jax.experimental.pallas import tpu as pltpu
from jax.experimental.pallas import tpu_sc as plsc

N = 10000
E = 320000
D = 128
EPS = 1e-5

NC = 2
NS = 16
K = 80
E_PER_CORE = E // NC
E_PER_SUB = E_PER_CORE // NS
N_CHUNKS = E_PER_SUB // K
NPAD = 10240
ROWS_PER_SUB = NPAD // NS

@functools.cache
def _build_deg_kernel():
    mesh = plsc.VectorSubcoreMesh(core_axis_name="c", subcore_axis_name="s")

    @functools.partial(
        pl.kernel,
        out_type=jax.ShapeDtypeStruct((NC, NPAD, D), jnp.float32),
        mesh=mesh,
        scratch_types=[
            pltpu.VMEM((N_CHUNKS, K), jnp.int32),
            pltpu.VMEM((K, D), jnp.float32),
            pltpu.SemaphoreType.DMA,
            pltpu.VMEM_SHARED((NPAD, D), jnp.float32),
        ],
    )
    def deg_kernel(dst_hbm, ones_hbm, zeros_hbm, out_hbm,
                   didx, ones_v, sem, acc):
        c = lax.axis_index("c")
        s = lax.axis_index("s")
        wid = c * NS + s
        pltpu.sync_copy(ones_hbm, ones_v)
        pltpu.sync_copy(zeros_hbm,
                        acc.at[pl.ds(s * ROWS_PER_SUB, ROWS_PER_SUB)])
        pltpu.sync_copy(dst_hbm.at[wid], didx)
        plsc.subcore_barrier()

        W = 8

        @pl.loop(0, N_CHUNKS)
        def _(i):
            pltpu.async_copy(ones_v, acc.at[didx.at[i]], sem, add=True)

            @pl.when(i >= W)
            def _():
                pltpu.make_async_copy(ones_v, acc.at[didx.at[i - W]],
                                      sem).wait()

        @pl.loop(N_CHUNKS - W, N_CHUNKS)
        def _(i):
            pltpu.make_async_copy(ones_v, acc.at[didx.at[i]], sem).wait()

        plsc.subcore_barrier()
        pltpu.sync_copy(
            acc.at[pl.ds(s * ROWS_PER_SUB, ROWS_PER_SUB)],
            out_hbm.at[c, pl.ds(s * ROWS_PER_SUB, ROWS_PER_SUB)],
        )

    return deg_kernel


@functools.cache
def _build_agg_kernel():
    mesh = plsc.VectorSubcoreMesh(core_axis_name="c", subcore_axis_name="s")

    @functools.partial(
        pl.kernel,
        out_type=jax.ShapeDtypeStruct((NC, NPAD, D), jnp.float32),
        mesh=mesh,
        scratch_types=[
            pltpu.VMEM((N_CHUNKS * K,), jnp.int32),
            pltpu.VMEM((N_CHUNKS, K), jnp.int32),
            pltpu.VMEM((K, D), jnp.float32),
            pltpu.VMEM((K, D), jnp.float32),
            pltpu.SemaphoreType.DMA,
            pltpu.SemaphoreType.DMA,
            pltpu.VMEM_SHARED((NPAD, D), jnp.float32),
        ],
    )
    def agg_kernel(h2_hbm, src_hbm, dst_hbm, zeros_hbm, out_hbm,
                   sidx, didx, rows0, rows1, sem0, sem1, acc):
        c = lax.axis_index("c")
        s = lax.axis_index("s")
        wid = c * NS + s
        pltpu.sync_copy(zeros_hbm,
                        acc.at[pl.ds(s * ROWS_PER_SUB, ROWS_PER_SUB)])
        pltpu.sync_copy(src_hbm.at[pl.ds(wid * N_CHUNKS * K, N_CHUNKS * K)],
                        sidx)
        pltpu.sync_copy(dst_hbm.at[wid], didx)
        plsc.subcore_barrier()

        def gather_start(i, rows, sem):
            pltpu.async_copy(h2_hbm.at[sidx.at[pl.ds(i * K, K)]], rows, sem)

        def gather_wait(i, rows, sem):
            pltpu.make_async_copy(h2_hbm.at[sidx.at[pl.ds(i * K, K)]],
                                  rows, sem).wait()

        def scatter(i, rows):
            pltpu.sync_copy(rows, acc.at[didx.at[i]], add=True)

        gather_start(0, rows0, sem0)

        @pl.loop(0, N_CHUNKS - 1, step=2)
        def _(i):
            gather_start(i + 1, rows1, sem1)
            gather_wait(i, rows0, sem0)
            scatter(i, rows0)
            gather_start(i + 2, rows0, sem0)
            gather_wait(i + 1, rows1, sem1)
            scatter(i + 1, rows1)

        gather_wait(N_CHUNKS - 1, rows0, sem0)
        scatter(N_CHUNKS - 1, rows0)

        plsc.subcore_barrier()
        pltpu.sync_copy(
            acc.at[pl.ds(s * ROWS_PER_SUB, ROWS_PER_SUB)],
            out_hbm.at[c, pl.ds(s * ROWS_PER_SUB, ROWS_PER_SUB)],
        )

    return agg_kernel


def _deg_kernel(dst3, ones_k, zerosD):
    return _build_deg_kernel()(dst3, ones_k, zerosD)[:, :N]


def _agg_kernel(h2, src, dst3, zerosD):
    return _build_agg_kernel()(h2, src, dst3, zerosD)[:, :N]


BN = 1000


def _mm_prep_body(p0_ref, p1_ref, x_ref, w_ref, oh_ref, od_ref):
    deg = p0_ref[:, :1] + p1_ref[:, :1] + 1.0
    dv = jnp.broadcast_to(lax.rsqrt(deg), (BN, D))
    od_ref[...] = dv
    oh_ref[...] = jnp.dot(x_ref[...], w_ref[...],
                          preferred_element_type=jnp.float32) * dv


def _mm_prep(degp, x, w):
    blk = pl.BlockSpec((BN, D), lambda i: (i, 0))
    return pl.pallas_call(
        _mm_prep_body,
        grid=(N // BN,),
        in_specs=[blk, blk, blk, pl.BlockSpec((D, D), lambda i: (0, 0))],
        out_specs=[blk, blk],
        out_shape=[jax.ShapeDtypeStruct((N, D), jnp.float32),
                   jax.ShapeDtypeStruct((N, D), jnp.float32)],
    )(degp[0], degp[1], x, w)


def _epi_core(a0, a1, h2, dv, b, xraw):
    pre = dv * (a0 + a1 + h2) + b
    r = jnp.maximum(pre, 0.0)
    mu = jnp.mean(r, axis=-1, keepdims=True)
    var = jnp.mean((r - mu) ** 2, axis=-1, keepdims=True)
    ln = (r - mu) * lax.rsqrt(var + EPS)
    return ln + xraw


def _epi_mm_body(a0_ref, a1_ref, h2_ref, dv_ref, b_ref, xr_ref, w_ref,
                 ox_ref, oh_ref):
    xn = _epi_core(a0_ref[...], a1_ref[...], h2_ref[...], dv_ref[...],
                   b_ref[...], xr_ref[...])
    ox_ref[...] = xn
    oh_ref[...] = jnp.dot(xn, w_ref[...],
                          preferred_element_type=jnp.float32) * dv_ref[...]


def _epi_mm(acc, h2, dinv2d, b, xraw, w_next):
    blk = pl.BlockSpec((BN, D), lambda i: (i, 0))
    return pl.pallas_call(
        _epi_mm_body,
        grid=(N // BN,),
        in_specs=[blk, blk, blk, blk,
                  pl.BlockSpec((1, D), lambda i: (0, 0)), blk,
                  pl.BlockSpec((D, D), lambda i: (0, 0))],
        out_specs=[blk, blk],
        out_shape=[jax.ShapeDtypeStruct((N, D), jnp.float32),
                   jax.ShapeDtypeStruct((N, D), jnp.float32)],
    )(acc[0], acc[1], h2, dinv2d, b.reshape(1, D), xraw, w_next)


def _epi_body(a0_ref, a1_ref, h2_ref, dv_ref, b_ref, xr_ref, ox_ref):
    ox_ref[...] = _epi_core(a0_ref[...], a1_ref[...], h2_ref[...],
                            dv_ref[...], b_ref[...], xr_ref[...])


def _epi(acc, h2, dinv2d, b, xraw):
    blk = pl.BlockSpec((BN, D), lambda i: (i, 0))
    return pl.pallas_call(
        _epi_body,
        grid=(N // BN,),
        in_specs=[blk, blk, blk, blk,
                  pl.BlockSpec((1, D), lambda i: (0, 0)), blk],
        out_specs=blk,
        out_shape=jax.ShapeDtypeStruct((N, D), jnp.float32),
    )(acc[0], acc[1], h2, dinv2d, b.reshape(1, D), xraw)


def kernel(x, edge, W0, b0, W1, b1, W2, b2):
    edge = edge.astype(jnp.int32)
    src = edge[0]
    dst = edge[1]
    dst3 = dst.reshape(NC * NS, N_CHUNKS, K)
    ones_k = jnp.ones((K, D), jnp.float32)
    zerosD = jnp.zeros((ROWS_PER_SUB, D), jnp.float32)

    degp = _deg_kernel(dst3, ones_k, zerosD)
    h2, dinv2d = _mm_prep(degp, x, W0)
    acc = _agg_kernel(h2, src, dst3, zerosD)
    x1, h2 = _epi_mm(acc, h2, dinv2d, b0, x, W1)

    acc = _agg_kernel(h2, src, dst3, zerosD)
    x2, h2 = _epi_mm(acc, h2, dinv2d, b1, x1, W2)

    acc = _agg_kernel(h2, src, dst3, zerosD)
    return _epi(acc, h2, dinv2d, b2, x2)
